# baseline (speedup 1.0000x reference)
import functools

import jax
import jax.numpy as jnp
from jax.experimental import pallas as pl


def _gcn_kernel(x_ref, w1_ref, pack_ref, out_ref, *, B, Nn, C):
    BN = B * Nn
    NN = Nn * Nn
    f32 = jnp.float32

    x = x_ref[...]
    w1 = w1_ref[...]
    adjf = pack_ref[0:1, :]
    b1 = pack_ref[1:2, :]
    b2 = pack_ref[2:3, 0:C]
    w2t = pack_ref[3:, :]

    k_mod = jax.lax.broadcasted_iota(jnp.int32, (NN, Nn), 0) % Nn
    j_idx = jax.lax.broadcasted_iota(jnp.int32, (NN, Nn), 1)
    S3 = (k_mod == j_idx).astype(f32)
    colsum = jnp.dot(adjf, S3, preferred_element_type=f32)
    deg = f32(B) * colsum + 1.0
    dis = jnp.where(deg > 0, jax.lax.rsqrt(deg), 0.0)

    xw = jnp.dot(x, w1, preferred_element_type=f32)

    rk = jax.lax.broadcasted_iota(jnp.int32, (BN, NN), 1)
    rp = jax.lax.broadcasted_iota(jnp.int32, (BN, NN), 0)
    S1 = ((rk % Nn) == (rp % Nn)).astype(f32)
    ck = jax.lax.broadcasted_iota(jnp.int32, (NN, BN), 0)
    cq = jax.lax.broadcasted_iota(jnp.int32, (NN, BN), 1)
    S2 = ((ck // Nn) == (cq % Nn)).astype(f32)
    TA = jnp.dot(S1 * adjf, S2, preferred_element_type=f32)

    pm = jax.lax.broadcasted_iota(jnp.int32, (BN, Nn), 0) % Nn
    pb = jax.lax.broadcasted_iota(jnp.int32, (BN, Nn), 1)
    C1 = (pm == pb).astype(f32)
    dis_p = jax.lax.dot_general(C1, dis, (((1,), (1,)), ((), ())),
                                preferred_element_type=f32)
    qa = jax.lax.broadcasted_iota(jnp.int32, (Nn, BN), 0)
    qm = jax.lax.broadcasted_iota(jnp.int32, (Nn, BN), 1) % Nn
    C2 = (qa == qm).astype(f32)
    dis_q = jnp.dot(dis, C2, preferred_element_type=f32)

    gp = jax.lax.broadcasted_iota(jnp.int32, (BN, BN), 0)
    gq = jax.lax.broadcasted_iota(jnp.int32, (BN, BN), 1)
    same_graph = ((gp // Nn) == (gq // Nn)).astype(f32)
    BD = same_graph * ((f32(B) * dis_p) * TA * dis_q)

    agg = jnp.dot(BD, xw, preferred_element_type=f32) + (dis_p * dis_p) * xw
    h = jnp.maximum(agg + b1, 0.0)

    bi = jax.lax.broadcasted_iota(jnp.int32, (B, BN), 0)
    pj = jax.lax.broadcasted_iota(jnp.int32, (B, BN), 1) // Nn
    P = (bi == pj).astype(f32) * (1.0 / f32(Nn))
    pooled = jnp.dot(P, h, preferred_element_type=f32)

    out = jax.lax.dot_general(pooled, w2t, (((1,), (1,)), ((), ())),
                              preferred_element_type=f32)
    out_ref[...] = out + b2


def kernel(batch, adj, W1, b1, W2, b2):
    B, Nn, F = batch.shape
    H = W1.shape[1]
    C = W2.shape[1]
    x = batch.reshape(B * Nn, F)
    pack = jnp.concatenate([
        adj.reshape(1, Nn * Nn),
        b1.reshape(1, H),
        jnp.pad(b2, (0, H - C)).reshape(1, H),
        W2.T,
    ], axis=0)
    body = functools.partial(_gcn_kernel, B=B, Nn=Nn, C=C)
    out = pl.pallas_call(
        body,
        out_shape=jax.ShapeDtypeStruct((B, C), batch.dtype),
    )(x, W1, pack)
    return out


# speedup vs baseline: 1.3126x; 1.3126x over previous
import functools

import jax
import jax.numpy as jnp
from jax.experimental import pallas as pl
from jax.experimental.pallas import tpu as pltpu


def _gcn_kernel(x_hbm, adj_hbm, w1_hbm, b1_hbm, w2_hbm, b2_hbm, out_ref,
                x_v, adj_v, w1_v, b1_v, w2_v, b2_v, sems, *, B, Nn):
    copies = [
        pltpu.make_async_copy(x_hbm, x_v, sems.at[0]),
        pltpu.make_async_copy(adj_hbm, adj_v, sems.at[1]),
        pltpu.make_async_copy(w1_hbm, w1_v, sems.at[2]),
        pltpu.make_async_copy(b1_hbm, b1_v, sems.at[3]),
        pltpu.make_async_copy(w2_hbm, w2_v, sems.at[4]),
        pltpu.make_async_copy(b2_hbm, b2_v, sems.at[5]),
    ]
    for c in copies:
        c.start()
    for c in copies:
        c.wait()

    BN = B * Nn
    f32 = jnp.float32

    x = x_v[...]
    adj = adj_v[...]
    w1 = w1_v[...]
    b1 = b1_v[...]
    w2 = w2_v[...]
    b2 = b2_v[...]

    colsum = jnp.sum(adj, axis=0, keepdims=True)
    deg = f32(B) * colsum + 1.0
    dis = jnp.where(deg > 0, jax.lax.rsqrt(deg), 0.0)

    xw = jnp.dot(x, w1, preferred_element_type=f32)

    p_mod = jax.lax.broadcasted_iota(jnp.int32, (BN, Nn), 0) % Nn
    b_idx = jax.lax.broadcasted_iota(jnp.int32, (BN, Nn), 1)
    C1 = (p_mod == b_idx).astype(f32)
    a_idx = jax.lax.broadcasted_iota(jnp.int32, (Nn, BN), 0)
    q_mod = jax.lax.broadcasted_iota(jnp.int32, (Nn, BN), 1) % Nn
    C2 = (a_idx == q_mod).astype(f32)

    t1 = jax.lax.dot_general(C1, adj, (((1,), (1,)), ((), ())),
                             preferred_element_type=f32)
    TA = jnp.dot(t1, C2, preferred_element_type=f32)

    dis_p = jax.lax.dot_general(C1, dis, (((1,), (1,)), ((), ())),
                                preferred_element_type=f32)
    dis_q = jnp.dot(dis, C2, preferred_element_type=f32)

    rp = jax.lax.broadcasted_iota(jnp.int32, (BN, BN), 0)
    cq = jax.lax.broadcasted_iota(jnp.int32, (BN, BN), 1)
    same_graph = ((rp // Nn) == (cq // Nn)).astype(f32)
    BD = same_graph * ((f32(B) * dis_p) * TA * dis_q)

    agg = jnp.dot(BD, xw, preferred_element_type=f32) + (dis_p * dis_p) * xw
    h = jnp.maximum(agg + b1, 0.0)

    bi = jax.lax.broadcasted_iota(jnp.int32, (B, BN), 0)
    pj = jax.lax.broadcasted_iota(jnp.int32, (B, BN), 1) // Nn
    P = (bi == pj).astype(f32) * (1.0 / f32(Nn))
    pooled = jnp.dot(P, h, preferred_element_type=f32)

    out_ref[...] = jnp.dot(pooled, w2, preferred_element_type=f32) + b2


def kernel(batch, adj, W1, b1, W2, b2):
    B, Nn, F = batch.shape
    H = W1.shape[1]
    C = W2.shape[1]
    x = batch.reshape(B * Nn, F)
    f32 = jnp.float32
    body = functools.partial(_gcn_kernel, B=B, Nn=Nn)
    out = pl.pallas_call(
        body,
        in_specs=[pl.BlockSpec(memory_space=pl.ANY)] * 6,
        out_specs=pl.BlockSpec(memory_space=pltpu.VMEM),
        out_shape=jax.ShapeDtypeStruct((B, C), batch.dtype),
        scratch_shapes=[
            pltpu.VMEM((B * Nn, F), f32),
            pltpu.VMEM((Nn, Nn), f32),
            pltpu.VMEM((F, H), f32),
            pltpu.VMEM((1, H), f32),
            pltpu.VMEM((H, C), f32),
            pltpu.VMEM((1, C), f32),
            pltpu.SemaphoreType.DMA((6,)),
        ],
    )(x, adj, W1, b1.reshape(1, H), W2, b2.reshape(1, C))
    return out


# speedup vs baseline: 1.3701x; 1.0438x over previous
import functools

import jax
import jax.numpy as jnp
from jax.experimental import pallas as pl


def _gcn_kernel(x_ref, adj_ref, w1_ref, b1_ref, w2_ref, b2_ref, out_ref,
                *, B, Nn):
    BN = B * Nn
    f32 = jnp.float32
    HIGHEST = jax.lax.Precision.HIGHEST

    x = x_ref[...]
    adj = adj_ref[...]
    w1 = w1_ref[...]
    b1 = b1_ref[...]
    w2 = w2_ref[...]
    b2 = b2_ref[...]

    colsum = jnp.sum(adj, axis=0, keepdims=True)
    deg = f32(B) * colsum + 1.0
    dis = jnp.where(deg > 0, jax.lax.rsqrt(deg), 0.0)

    xw = jnp.dot(x, w1, preferred_element_type=f32)

    p_mod = jax.lax.broadcasted_iota(jnp.int32, (BN, Nn), 0) % Nn
    b_idx = jax.lax.broadcasted_iota(jnp.int32, (BN, Nn), 1)
    C1 = (p_mod == b_idx).astype(f32)
    a_idx = jax.lax.broadcasted_iota(jnp.int32, (Nn, BN), 0)
    q_mod = jax.lax.broadcasted_iota(jnp.int32, (Nn, BN), 1) % Nn
    C2 = (a_idx == q_mod).astype(f32)

    t1 = jax.lax.dot_general(C1, adj, (((1,), (1,)), ((), ())),
                             preferred_element_type=f32)
    TA = jnp.dot(t1, C2, preferred_element_type=f32)

    dis_p = jax.lax.dot_general(C1, dis, (((1,), (1,)), ((), ())),
                                preferred_element_type=f32)
    dis_q = jnp.dot(dis, C2, preferred_element_type=f32)

    z_div = jax.lax.broadcasted_iota(jnp.int32, (BN, B), 0) // Nn
    z_g = jax.lax.broadcasted_iota(jnp.int32, (BN, B), 1)
    Z = (z_div == z_g).astype(f32)
    w_g = jax.lax.broadcasted_iota(jnp.int32, (B, BN), 0)
    w_div = jax.lax.broadcasted_iota(jnp.int32, (B, BN), 1) // Nn
    Wm = (w_g == w_div).astype(f32)

    same_graph = jnp.dot(Z, Wm, preferred_element_type=f32)
    BD = same_graph * ((f32(B) * dis_p) * TA * dis_q)

    agg = jnp.dot(BD, xw, preferred_element_type=f32) + (dis_p * dis_p) * xw
    h = jnp.maximum(agg + b1, 0.0)

    P = Wm * (1.0 / f32(Nn))
    pooled = jnp.dot(P, h, preferred_element_type=f32)

    out_ref[...] = jnp.dot(pooled, w2, preferred_element_type=f32) + b2


def kernel(batch, adj, W1, b1, W2, b2):
    B, Nn, F = batch.shape
    H = W1.shape[1]
    C = W2.shape[1]
    x = batch.reshape(B * Nn, F)
    body = functools.partial(_gcn_kernel, B=B, Nn=Nn)
    out = pl.pallas_call(
        body,
        out_shape=jax.ShapeDtypeStruct((B, C), batch.dtype),
    )(x, adj, W1, b1.reshape(1, H), W2, b2.reshape(1, C))
    return out
